# baseline (speedup 1.0000x reference)
import jax
import jax.numpy as jnp
from jax.experimental import pallas as pl
from jax.experimental.pallas import tpu as pltpu

VOCAB_SIZE = 25107
EMB_DIM = 100
CTX_LEN = 5
BATCH = 16
H1 = 300
H2 = 300

VOCAB_TILE = 2048
NUM_FULL_TILES = VOCAB_SIZE // VOCAB_TILE
TAIL = VOCAB_SIZE - NUM_FULL_TILES * VOCAB_TILE
NBUF = 4


def _mlp_kernel(x_smem, emb_hbm, w1_ref, b1_ref, w2_ref, b2_ref, w3_hbm,
                b3_ref, out_ref, wbuf, ebuf, bufs, tail_buf, gsem, sems,
                tail_sem):
    def start_fetch(i):
        pltpu.make_async_copy(
            w3_hbm.at[:, pl.ds(i * VOCAB_TILE, VOCAB_TILE)],
            bufs.at[i % NBUF],
            sems.at[i % NBUF],
        ).start()

    tail_copy = pltpu.make_async_copy(
        w3_hbm.at[:, pl.ds(NUM_FULL_TILES * VOCAB_TILE, TAIL)],
        tail_buf,
        tail_sem,
    )
    tail_copy.start()
    for i in range(NBUF):
        start_fetch(i)

    gathers = []
    for b in range(BATCH):
        for c in range(CTX_LEN):
            start = x_smem[b, c] * EMB_DIM
            win = (start // 128) * 128
            g = pltpu.make_async_copy(
                emb_hbm.at[pl.ds(win, 256)],
                wbuf.at[c, b], gsem)
            g.start()
            gathers.append(g)
    for g in gathers:
        g.wait()
    for b in range(BATCH):
        for c in range(CTX_LEN):
            start = x_smem[b, c] * EMB_DIM
            off = start % 128
            row = pltpu.roll(wbuf[c, pl.ds(b, 1), :], -off, 1)
            ebuf[c, pl.ds(b, 1), :] = row[:, :EMB_DIM]

    h1 = b1_ref[...][None, :]
    for c in range(CTX_LEN):
        h1 = h1 + jnp.dot(ebuf[c], w1_ref[c],
                          preferred_element_type=jnp.float32)
    h1 = jnp.maximum(h1, 0.0)
    h2 = jnp.maximum(
        jnp.dot(h1, w2_ref[...],
                preferred_element_type=jnp.float32) + b2_ref[...][None, :],
        0.0)

    for i in range(NUM_FULL_TILES):
        pltpu.make_async_copy(
            w3_hbm.at[:, pl.ds(i * VOCAB_TILE, VOCAB_TILE)],
            bufs.at[i % NBUF],
            sems.at[i % NBUF],
        ).wait()
        tile = jnp.dot(h2, bufs[i % NBUF],
                       preferred_element_type=jnp.float32)
        if i + NBUF < NUM_FULL_TILES:
            start_fetch(i + NBUF)
        out_ref[:, pl.ds(i * VOCAB_TILE, VOCAB_TILE)] = (
            tile + b3_ref[pl.ds(i * VOCAB_TILE, VOCAB_TILE)][None, :])

    tail_copy.wait()
    base = NUM_FULL_TILES * VOCAB_TILE
    tail = jnp.dot(h2, tail_buf[...], preferred_element_type=jnp.float32)
    out_ref[:, pl.ds(base, TAIL)] = tail + b3_ref[pl.ds(base, TAIL)][None, :]


def kernel(x, emb, W1, b1, W2, b2, W3, b3):
    return pl.pallas_call(
        _mlp_kernel,
        in_specs=[
            pl.BlockSpec(memory_space=pltpu.SMEM),
            pl.BlockSpec(memory_space=pl.ANY),
            pl.BlockSpec(memory_space=pltpu.VMEM),
            pl.BlockSpec(memory_space=pltpu.VMEM),
            pl.BlockSpec(memory_space=pltpu.VMEM),
            pl.BlockSpec(memory_space=pltpu.VMEM),
            pl.BlockSpec(memory_space=pl.ANY),
            pl.BlockSpec(memory_space=pltpu.VMEM),
        ],
        out_specs=pl.BlockSpec(memory_space=pltpu.VMEM),
        out_shape=jax.ShapeDtypeStruct((BATCH, VOCAB_SIZE), jnp.float32),
        scratch_shapes=[
            pltpu.VMEM((CTX_LEN, BATCH, 256), jnp.float32),
            pltpu.VMEM((CTX_LEN, BATCH, EMB_DIM), jnp.float32),
            pltpu.VMEM((NBUF, H2, VOCAB_TILE), jnp.float32),
            pltpu.VMEM((H2, TAIL), jnp.float32),
            pltpu.SemaphoreType.DMA,
            pltpu.SemaphoreType.DMA((NBUF,)),
            pltpu.SemaphoreType.DMA,
        ],
    )(x, emb.reshape(-1), W1.reshape(CTX_LEN, EMB_DIM, H1), b1, W2, b2, W3,
      b3)


# speedup vs baseline: 3.3980x; 3.3980x over previous
import jax
import jax.numpy as jnp
from jax.experimental import pallas as pl
from jax.experimental.pallas import tpu as pltpu

VOCAB_SIZE = 25107
EMB_DIM = 100
CTX_LEN = 5
BATCH = 16
H1 = 300
H2 = 300

VOCAB_TILE = 2048
NUM_FULL_TILES = VOCAB_SIZE // VOCAB_TILE
TAIL = VOCAB_SIZE - NUM_FULL_TILES * VOCAB_TILE
NBUF = 4


def _mlp_kernel(x_smem, emb_hbm, w1_ref, b1_ref, w2_ref, b2_ref, w3_hbm,
                b3_ref, out_ref, ebuf, bufs, tail_buf, gsem, sems, tail_sem):
    def start_fetch(i):
        pltpu.make_async_copy(
            w3_hbm.at[:, pl.ds(i * VOCAB_TILE, VOCAB_TILE)],
            bufs.at[i % NBUF],
            sems.at[i % NBUF],
        ).start()

    tail_copy = pltpu.make_async_copy(
        w3_hbm.at[:, pl.ds(NUM_FULL_TILES * VOCAB_TILE, TAIL)],
        tail_buf,
        tail_sem,
    )
    tail_copy.start()
    for i in range(NBUF):
        start_fetch(i)

    gathers = []
    for b in range(BATCH):
        for c in range(CTX_LEN):
            g = pltpu.make_async_copy(
                emb_hbm.at[pl.ds(x_smem[b, c], 1), :],
                ebuf.at[c, pl.ds(b, 1), :], gsem)
            g.start()
            gathers.append(g)
    for g in gathers:
        g.wait()

    w1v = w1_ref[...]
    h1 = b1_ref[...][None, :]
    for c in range(CTX_LEN):
        h1 = h1 + jnp.dot(ebuf[c], w1v[c * EMB_DIM:(c + 1) * EMB_DIM, :],
                          preferred_element_type=jnp.float32)
    h1 = jnp.maximum(h1, 0.0)
    h2 = jnp.maximum(
        jnp.dot(h1, w2_ref[...],
                preferred_element_type=jnp.float32) + b2_ref[...][None, :],
        0.0)

    for i in range(NUM_FULL_TILES):
        pltpu.make_async_copy(
            w3_hbm.at[:, pl.ds(i * VOCAB_TILE, VOCAB_TILE)],
            bufs.at[i % NBUF],
            sems.at[i % NBUF],
        ).wait()
        tile = jnp.dot(h2, bufs[i % NBUF],
                       preferred_element_type=jnp.float32)
        if i + NBUF < NUM_FULL_TILES:
            start_fetch(i + NBUF)
        out_ref[:, pl.ds(i * VOCAB_TILE, VOCAB_TILE)] = (
            tile + b3_ref[pl.ds(i * VOCAB_TILE, VOCAB_TILE)][None, :])

    tail_copy.wait()
    base = NUM_FULL_TILES * VOCAB_TILE
    tail = jnp.dot(h2, tail_buf[...], preferred_element_type=jnp.float32)
    out_ref[:, pl.ds(base, TAIL)] = tail + b3_ref[pl.ds(base, TAIL)][None, :]


def kernel(x, emb, W1, b1, W2, b2, W3, b3):
    return pl.pallas_call(
        _mlp_kernel,
        in_specs=[
            pl.BlockSpec(memory_space=pltpu.SMEM),
            pl.BlockSpec(memory_space=pl.ANY),
            pl.BlockSpec(memory_space=pltpu.VMEM),
            pl.BlockSpec(memory_space=pltpu.VMEM),
            pl.BlockSpec(memory_space=pltpu.VMEM),
            pl.BlockSpec(memory_space=pltpu.VMEM),
            pl.BlockSpec(memory_space=pl.ANY),
            pl.BlockSpec(memory_space=pltpu.VMEM),
        ],
        out_specs=pl.BlockSpec(memory_space=pltpu.VMEM),
        out_shape=jax.ShapeDtypeStruct((BATCH, VOCAB_SIZE), jnp.float32),
        scratch_shapes=[
            pltpu.VMEM((CTX_LEN, BATCH, EMB_DIM), jnp.float32),
            pltpu.VMEM((NBUF, H2, VOCAB_TILE), jnp.float32),
            pltpu.VMEM((H2, TAIL), jnp.float32),
            pltpu.SemaphoreType.DMA,
            pltpu.SemaphoreType.DMA((NBUF,)),
            pltpu.SemaphoreType.DMA,
        ],
    )(x, emb, W1, b1, W2, b2, W3, b3)


# speedup vs baseline: 4.7700x; 1.4038x over previous
import jax
import jax.numpy as jnp
from jax.experimental import pallas as pl
from jax.experimental.pallas import tpu as pltpu

VOCAB_SIZE = 25107
EMB_DIM = 100
CTX_LEN = 5
BATCH = 16
H1 = 300
H2 = 300

VOCAB_TILE = 2048
NUM_FULL_TILES = VOCAB_SIZE // VOCAB_TILE
TAIL = VOCAB_SIZE - NUM_FULL_TILES * VOCAB_TILE
NBUF = 4
LANE = 128


def _mlp_kernel(xt_smem, embt_hbm, w1t_ref, b1_ref, w2_ref, b2_ref, w3_hbm,
                b3_ref, out_ref, wbuf, ebuf_t, bufs, tail_buf, gsem, sems,
                tail_sem):
    def start_fetch(i):
        pltpu.make_async_copy(
            w3_hbm.at[:, pl.ds(i * VOCAB_TILE, VOCAB_TILE)],
            bufs.at[i % NBUF],
            sems.at[i % NBUF],
        ).start()

    tail_copy = pltpu.make_async_copy(
        w3_hbm.at[:, pl.ds(NUM_FULL_TILES * VOCAB_TILE, TAIL)],
        tail_buf,
        tail_sem,
    )
    tail_copy.start()
    for i in range(NBUF):
        start_fetch(i)

    gathers = []
    for b in range(BATCH):
        for c in range(CTX_LEN):
            win = (xt_smem[c, b] // LANE) * LANE
            g = pltpu.make_async_copy(
                embt_hbm.at[:, pl.ds(win, LANE)],
                wbuf.at[c, b], gsem)
            g.start()
            gathers.append(g)
    for g in gathers:
        g.wait()

    for b in range(BATCH):
        for c in range(CTX_LEN):
            off = xt_smem[c, b] % LANE
            rolled = pltpu.roll(wbuf[c, b], -off, 1)
            ebuf_t[c, :, pl.ds(b, 1)] = rolled[:, :1]

    w1t = w1t_ref[...]
    h1t = jnp.dot(w1t[:, 0:EMB_DIM], ebuf_t[0],
                  preferred_element_type=jnp.float32)
    for c in range(1, CTX_LEN):
        h1t = h1t + jnp.dot(w1t[:, c * EMB_DIM:(c + 1) * EMB_DIM], ebuf_t[c],
                            preferred_element_type=jnp.float32)
    h1 = jnp.maximum(h1t.T + b1_ref[...][None, :], 0.0)
    h2 = jnp.maximum(
        jnp.dot(h1, w2_ref[...],
                preferred_element_type=jnp.float32) + b2_ref[...][None, :],
        0.0)

    for i in range(NUM_FULL_TILES):
        pltpu.make_async_copy(
            w3_hbm.at[:, pl.ds(i * VOCAB_TILE, VOCAB_TILE)],
            bufs.at[i % NBUF],
            sems.at[i % NBUF],
        ).wait()
        tile = jnp.dot(h2, bufs[i % NBUF],
                       preferred_element_type=jnp.float32)
        if i + NBUF < NUM_FULL_TILES:
            start_fetch(i + NBUF)
        out_ref[:, pl.ds(i * VOCAB_TILE, VOCAB_TILE)] = (
            tile + b3_ref[pl.ds(i * VOCAB_TILE, VOCAB_TILE)][None, :])

    tail_copy.wait()
    base = NUM_FULL_TILES * VOCAB_TILE
    tail = jnp.dot(h2, tail_buf[...], preferred_element_type=jnp.float32)
    out_ref[:, pl.ds(base, TAIL)] = tail + b3_ref[pl.ds(base, TAIL)][None, :]


def kernel(x, emb, W1, b1, W2, b2, W3, b3):
    return pl.pallas_call(
        _mlp_kernel,
        in_specs=[
            pl.BlockSpec(memory_space=pltpu.SMEM),
            pl.BlockSpec(memory_space=pl.ANY),
            pl.BlockSpec(memory_space=pltpu.VMEM),
            pl.BlockSpec(memory_space=pltpu.VMEM),
            pl.BlockSpec(memory_space=pltpu.VMEM),
            pl.BlockSpec(memory_space=pltpu.VMEM),
            pl.BlockSpec(memory_space=pl.ANY),
            pl.BlockSpec(memory_space=pltpu.VMEM),
        ],
        out_specs=pl.BlockSpec(memory_space=pltpu.VMEM),
        out_shape=jax.ShapeDtypeStruct((BATCH, VOCAB_SIZE), jnp.float32),
        scratch_shapes=[
            pltpu.VMEM((CTX_LEN, BATCH, EMB_DIM, LANE), jnp.float32),
            pltpu.VMEM((CTX_LEN, EMB_DIM, BATCH), jnp.float32),
            pltpu.VMEM((NBUF, H2, VOCAB_TILE), jnp.float32),
            pltpu.VMEM((H2, TAIL), jnp.float32),
            pltpu.SemaphoreType.DMA,
            pltpu.SemaphoreType.DMA((NBUF,)),
            pltpu.SemaphoreType.DMA,
        ],
    )(x.T, emb.T, W1.T, b1, W2, b2, W3, b3)


# speedup vs baseline: 5.8544x; 1.2273x over previous
import jax
import jax.numpy as jnp
from jax.experimental import pallas as pl
from jax.experimental.pallas import tpu as pltpu

VOCAB_SIZE = 25107
EMB_DIM = 100
CTX_LEN = 5
BATCH = 16
H1 = 300
H2 = 300

VOCAB_TILE = 2048
NUM_FULL_TILES = VOCAB_SIZE // VOCAB_TILE
TAIL = VOCAB_SIZE - NUM_FULL_TILES * VOCAB_TILE
NBUF = 4
LANE = 128


def _mlp_kernel(xt_smem, xt_vmem, embt_hbm, w1t_ref, b1_ref, w2_ref, b2_ref,
                w3_hbm, b3_ref, out_ref, wbuf, bufs, tail_buf, gsem, sems,
                tail_sem):
    def start_fetch(i):
        pltpu.make_async_copy(
            w3_hbm.at[:, pl.ds(i * VOCAB_TILE, VOCAB_TILE)],
            bufs.at[i % NBUF],
            sems.at[i % NBUF],
        ).start()

    tail_copy = pltpu.make_async_copy(
        w3_hbm.at[:, pl.ds(NUM_FULL_TILES * VOCAB_TILE, TAIL)],
        tail_buf,
        tail_sem,
    )
    tail_copy.start()
    for i in range(NBUF):
        start_fetch(i)

    gathers = []
    for b in range(BATCH):
        for c in range(CTX_LEN):
            win = (xt_smem[c, b] // LANE) * LANE
            g = pltpu.make_async_copy(
                embt_hbm.at[:, pl.ds(win, LANE)],
                wbuf.at[b, c], gsem)
            g.start()
            gathers.append(g)
    for g in gathers:
        g.wait()

    offs = jnp.transpose(xt_vmem[...], (1, 0)) % LANE
    lane_iota = jax.lax.broadcasted_iota(jnp.int32, (BATCH, 1, LANE), 2)

    w1t = w1t_ref[...]
    h1 = b1_ref[...][None, :]
    for c in range(CTX_LEN):
        sel = (lane_iota == offs[:, c][:, None, None]).astype(jnp.float32)
        e_c = jnp.sum(wbuf[:, c] * sel, axis=2)
        h1 = h1 + jax.lax.dot_general(
            e_c, w1t[:, c * EMB_DIM:(c + 1) * EMB_DIM],
            (((1,), (1,)), ((), ())),
            preferred_element_type=jnp.float32)
    h1 = jnp.maximum(h1, 0.0)
    h2 = jnp.maximum(
        jnp.dot(h1, w2_ref[...],
                preferred_element_type=jnp.float32) + b2_ref[...][None, :],
        0.0)

    for i in range(NUM_FULL_TILES):
        pltpu.make_async_copy(
            w3_hbm.at[:, pl.ds(i * VOCAB_TILE, VOCAB_TILE)],
            bufs.at[i % NBUF],
            sems.at[i % NBUF],
        ).wait()
        tile = jnp.dot(h2, bufs[i % NBUF],
                       preferred_element_type=jnp.float32)
        if i + NBUF < NUM_FULL_TILES:
            start_fetch(i + NBUF)
        out_ref[:, pl.ds(i * VOCAB_TILE, VOCAB_TILE)] = (
            tile + b3_ref[pl.ds(i * VOCAB_TILE, VOCAB_TILE)][None, :])

    tail_copy.wait()
    base = NUM_FULL_TILES * VOCAB_TILE
    tail = jnp.dot(h2, tail_buf[...], preferred_element_type=jnp.float32)
    out_ref[:, pl.ds(base, TAIL)] = tail + b3_ref[pl.ds(base, TAIL)][None, :]


def kernel(x, emb, W1, b1, W2, b2, W3, b3):
    return pl.pallas_call(
        _mlp_kernel,
        in_specs=[
            pl.BlockSpec(memory_space=pltpu.SMEM),
            pl.BlockSpec(memory_space=pltpu.VMEM),
            pl.BlockSpec(memory_space=pl.ANY),
            pl.BlockSpec(memory_space=pltpu.VMEM),
            pl.BlockSpec(memory_space=pltpu.VMEM),
            pl.BlockSpec(memory_space=pltpu.VMEM),
            pl.BlockSpec(memory_space=pltpu.VMEM),
            pl.BlockSpec(memory_space=pl.ANY),
            pl.BlockSpec(memory_space=pltpu.VMEM),
        ],
        out_specs=pl.BlockSpec(memory_space=pltpu.VMEM),
        out_shape=jax.ShapeDtypeStruct((BATCH, VOCAB_SIZE), jnp.float32),
        scratch_shapes=[
            pltpu.VMEM((BATCH, CTX_LEN, EMB_DIM, LANE), jnp.float32),
            pltpu.VMEM((NBUF, H2, VOCAB_TILE), jnp.float32),
            pltpu.VMEM((H2, TAIL), jnp.float32),
            pltpu.SemaphoreType.DMA,
            pltpu.SemaphoreType.DMA((NBUF,)),
            pltpu.SemaphoreType.DMA,
        ],
    )(x.T, x.T, emb.T, W1.T, b1, W2, b2, W3, b3)


# speedup vs baseline: 5.9937x; 1.0238x over previous
import jax
import jax.numpy as jnp
from jax.experimental import pallas as pl
from jax.experimental.pallas import tpu as pltpu

VOCAB_SIZE = 25107
EMB_DIM = 100
CTX_LEN = 5
BATCH = 16
H1 = 300
H2 = 300

VOCAB_TILE = 2048
NUM_FULL_TILES = VOCAB_SIZE // VOCAB_TILE
TAIL = VOCAB_SIZE - NUM_FULL_TILES * VOCAB_TILE
NBUF = 6
LANE = 128


def _mlp_kernel(xt_smem, xt_vmem, embt_hbm, w1t_ref, b1_ref, w2_ref, b2_ref,
                w3_hbm, b3_ref, out_ref, wbuf, bufs, tail_buf, gsem, sems,
                tail_sem):
    def start_fetch(i):
        pltpu.make_async_copy(
            w3_hbm.at[:, pl.ds(i * VOCAB_TILE, VOCAB_TILE)],
            bufs.at[i % NBUF],
            sems.at[i % NBUF],
        ).start()

    tail_copy = pltpu.make_async_copy(
        w3_hbm.at[:, pl.ds(NUM_FULL_TILES * VOCAB_TILE, TAIL)],
        tail_buf,
        tail_sem,
    )
    tail_copy.start()
    for i in range(NBUF):
        start_fetch(i)

    gathers = []
    for b in range(BATCH):
        for c in range(CTX_LEN):
            win = (xt_smem[c, b] // LANE) * LANE
            g = pltpu.make_async_copy(
                embt_hbm.at[:, pl.ds(win, LANE)],
                wbuf.at[b, c], gsem)
            g.start()
            gathers.append(g)
    for g in gathers:
        g.wait()

    offs = jnp.transpose(xt_vmem[...], (1, 0)) % LANE
    lane_iota = jax.lax.broadcasted_iota(jnp.int32, (BATCH, 1, LANE), 2)

    w1t = w1t_ref[...]
    h1 = b1_ref[...][None, :]
    for c in range(CTX_LEN):
        sel = (lane_iota == offs[:, c][:, None, None]).astype(jnp.float32)
        e_c = jnp.sum(wbuf[:, c] * sel, axis=2)
        h1 = h1 + jax.lax.dot_general(
            e_c, w1t[:, c * EMB_DIM:(c + 1) * EMB_DIM],
            (((1,), (1,)), ((), ())),
            preferred_element_type=jnp.float32)
    h1 = jnp.maximum(h1, 0.0)
    h2 = jnp.maximum(
        jnp.dot(h1, w2_ref[...],
                preferred_element_type=jnp.float32) + b2_ref[...][None, :],
        0.0)

    for i in range(NUM_FULL_TILES):
        pltpu.make_async_copy(
            w3_hbm.at[:, pl.ds(i * VOCAB_TILE, VOCAB_TILE)],
            bufs.at[i % NBUF],
            sems.at[i % NBUF],
        ).wait()
        tile = jnp.dot(h2, bufs[i % NBUF],
                       preferred_element_type=jnp.float32)
        if i + NBUF < NUM_FULL_TILES:
            start_fetch(i + NBUF)
        out_ref[:, pl.ds(i * VOCAB_TILE, VOCAB_TILE)] = (
            tile + b3_ref[pl.ds(i * VOCAB_TILE, VOCAB_TILE)][None, :])

    tail_copy.wait()
    base = NUM_FULL_TILES * VOCAB_TILE
    tail = jnp.dot(h2, tail_buf[...], preferred_element_type=jnp.float32)
    out_ref[:, pl.ds(base, TAIL)] = tail + b3_ref[pl.ds(base, TAIL)][None, :]


def kernel(x, emb, W1, b1, W2, b2, W3, b3):
    return pl.pallas_call(
        _mlp_kernel,
        in_specs=[
            pl.BlockSpec(memory_space=pltpu.SMEM),
            pl.BlockSpec(memory_space=pltpu.VMEM),
            pl.BlockSpec(memory_space=pl.ANY),
            pl.BlockSpec(memory_space=pltpu.VMEM),
            pl.BlockSpec(memory_space=pltpu.VMEM),
            pl.BlockSpec(memory_space=pltpu.VMEM),
            pl.BlockSpec(memory_space=pltpu.VMEM),
            pl.BlockSpec(memory_space=pl.ANY),
            pl.BlockSpec(memory_space=pltpu.VMEM),
        ],
        out_specs=pl.BlockSpec(memory_space=pltpu.VMEM),
        out_shape=jax.ShapeDtypeStruct((BATCH, VOCAB_SIZE), jnp.float32),
        scratch_shapes=[
            pltpu.VMEM((BATCH, CTX_LEN, EMB_DIM, LANE), jnp.float32),
            pltpu.VMEM((NBUF, H2, VOCAB_TILE), jnp.float32),
            pltpu.VMEM((H2, TAIL), jnp.float32),
            pltpu.SemaphoreType.DMA,
            pltpu.SemaphoreType.DMA((NBUF,)),
            pltpu.SemaphoreType.DMA,
        ],
    )(x.T, x.T, emb.T, W1.T, b1, W2, b2, W3, b3)
